# Initial kernel scaffold; baseline (speedup 1.0000x reference)
import jax
import jax.numpy as jnp
from jax.experimental import pallas as pl


def kernel(xyz, features, W1, g1, b1, W2, g2, b2, Wscore, Wmlp, g3, b3, Wshort, gs, bs):
    raise NotImplementedError("write your pallas kernel here")


# speedup vs baseline: 5.3211x; 5.3211x over previous
import functools

import jax
import jax.numpy as jnp
from jax import lax
from jax.experimental import pallas as pl
from jax.experimental.pallas import tpu as pltpu
from jax.experimental.pallas import tpu_sc as plsc

B, N, K = 4, 4096, 16
D_IN, D_OUT = 64, 128
D_MID = D_OUT // 2
D_AP = D_MID + D_IN
EPS = 1e-5

TQ = 256
NTQ = N // TQ
TG = 2048
QT = TG // K
NTG = N // QT
ROWS = B * N * K
MROWS = B * N

_HIGH = lax.Precision.HIGHEST


def _dot(a, b, dims):
    return lax.dot_general(a, b, (dims, ((), ())), precision=_HIGH,
                           preferred_element_type=jnp.float32)


def _knn_kernel(q_ref, a_ref, sqq_ref, sqa_ref, idx_ref):
    b = pl.program_id(0)
    cross = lax.dot_general(q_ref[0, 0], a_ref[0, 0], (((1,), (1,)), ((), ())),
                            preferred_element_type=jnp.float32)
    d = sqq_ref[0, 0] + sqa_ref[0] - 2.0 * cross
    col = lax.broadcasted_iota(jnp.int32, (TQ, N), 1)
    base = (b * N).astype(jnp.int32)
    picks = []
    for _ in range(K):
        m = jnp.min(d, axis=1, keepdims=True)
        am = jnp.min(jnp.where(d == m, col, N), axis=1, keepdims=True)
        picks.append(am + base)
        d = jnp.where(col == am, jnp.float32(jnp.inf), d)
    idx_ref[0, 0] = jnp.concatenate(picks, axis=1)


def _knn(xyz):
    sq = jnp.sum(xyz * xyz, -1)
    out = pl.pallas_call(
        _knn_kernel,
        grid=(B, NTQ),
        in_specs=[pl.BlockSpec((1, 1, TQ, 3), lambda b, t: (b, t, 0, 0)),
                  pl.BlockSpec((1, 1, N, 3), lambda b, t: (b, 0, 0, 0)),
                  pl.BlockSpec((1, 1, TQ, 1), lambda b, t: (b, t, 0, 0)),
                  pl.BlockSpec((1, 1, N), lambda b, t: (b, 0, 0))],
        out_specs=pl.BlockSpec((1, 1, TQ, K), lambda b, t: (b, t, 0, 0)),
        out_shape=jax.ShapeDtypeStruct((B, NTQ, TQ, K), jnp.int32),
    )(xyz.reshape(B, NTQ, TQ, 3), xyz.reshape(B, 1, N, 3),
      sq.reshape(B, NTQ, TQ, 1), sq.reshape(B, 1, N))
    return out.reshape(ROWS)


_CH = 128
_RPW = ROWS // 32
_NCH = _RPW // _CH


def _sc_gather(tab, idx2d):
    info = plsc.get_sparse_core_info()
    nc = info.num_cores
    mesh = plsc.VectorSubcoreMesh(core_axis_name="c", subcore_axis_name="s")

    @functools.partial(
        pl.kernel,
        mesh=mesh,
        out_type=jax.ShapeDtypeStruct((ROWS, D_AP), jnp.float32),
        scratch_types=[
            pltpu.VMEM((_NCH, _CH), jnp.int32),
            pltpu.VMEM((2, _CH, D_AP), jnp.float32),
            pltpu.SemaphoreType.DMA,
            pltpu.SemaphoreType.DMA,
        ],
    )
    def gather_k(tab_hbm, idx_hbm, out_f, idx_v, fv, s1, s2):
        wid = lax.axis_index("s") * nc + lax.axis_index("c")
        pltpu.sync_copy(idx_hbm.at[pl.ds(wid * _NCH, _NCH)], idx_v)
        base = wid * _RPW
        pltpu.async_copy(tab_hbm.at[idx_v.at[0]], fv.at[0], s1)

        @pl.loop(0, _NCH, step=2)
        def _chunk(c):
            pltpu.async_copy(tab_hbm.at[idx_v.at[c + 1]], fv.at[1], s2)
            pltpu.make_async_copy(tab_hbm.at[pl.ds(0, _CH)], fv.at[0],
                                  s1).wait()
            pltpu.sync_copy(fv.at[0], out_f.at[pl.ds(base + c * _CH, _CH)])

            @pl.when(c + 2 < _NCH)
            def _():
                pltpu.async_copy(tab_hbm.at[idx_v.at[c + 2]], fv.at[0], s1)

            pltpu.make_async_copy(tab_hbm.at[pl.ds(0, _CH)], fv.at[1],
                                  s2).wait()
            pltpu.sync_copy(fv.at[1],
                            out_f.at[pl.ds(base + (c + 1) * _CH, _CH)])

    return gather_k(tab, idx2d)


def _stats_kernel(x_ref, s1_ref, s2_ref, s1a, s2a):
    i = pl.program_id(0)
    x = x_ref[...]

    @pl.when(i == 0)
    def _():
        s1a[...] = jnp.zeros_like(s1a)
        s2a[...] = jnp.zeros_like(s2a)

    s1a[...] += jnp.sum(x, axis=0, keepdims=True)
    s2a[...] += _dot(x, x, ((0,), (0,)))

    @pl.when(i == pl.num_programs(0) - 1)
    def _():
        s1_ref[...] = s1a[...]
        s2_ref[...] = s2a[...]


def _moments(x2d, tile):
    rows, d = x2d.shape
    grid = rows // tile
    s1, s2 = pl.pallas_call(
        _stats_kernel,
        grid=(grid,),
        in_specs=[pl.BlockSpec((tile, d), lambda i: (i, 0))],
        out_specs=[pl.BlockSpec((1, d), lambda i: (0, 0)),
                   pl.BlockSpec((d, d), lambda i: (0, 0))],
        out_shape=[jax.ShapeDtypeStruct((1, d), jnp.float32),
                   jax.ShapeDtypeStruct((d, d), jnp.float32)],
        scratch_shapes=[pltpu.VMEM((1, d), jnp.float32),
                        pltpu.VMEM((d, d), jnp.float32)],
    )(x2d)
    return s1, s2


def _bn_fold(s1, s2, m_count, w, g, bias):
    mean_x = s1[0] / m_count
    mean_y = w @ mean_x
    ey2 = jnp.einsum("ij,jk,ik->i", w, s2 / m_count, w)
    var_y = ey2 - mean_y * mean_y
    a = g / jnp.sqrt(var_y + EPS)
    c = bias - a * mean_y
    return a.reshape(1, -1), c.reshape(1, -1)


def _spatial(nx_ref, q_ref):
    nx = nx_ref[...][:, 0:3]
    q = q_ref[...]
    rel = nx - q
    dist = jnp.sum(rel * rel, axis=1, keepdims=True)
    return jnp.concatenate([q, nx, rel, dist], axis=1)


def _sp_stats_kernel(nx_ref, q_ref, s1_ref, s2_ref, s1a, s2a):
    i = pl.program_id(0)
    sp = _spatial(nx_ref, q_ref)

    @pl.when(i == 0)
    def _():
        s1a[...] = jnp.zeros_like(s1a)
        s2a[...] = jnp.zeros_like(s2a)

    s1a[...] += jnp.sum(sp, axis=0, keepdims=True)
    s2a[...] += _dot(sp, sp, ((0,), (0,)))

    @pl.when(i == pl.num_programs(0) - 1)
    def _():
        s1_ref[...] = s1a[...]
        s2_ref[...] = s2a[...]


def _h_stats_kernel(nx_ref, q_ref, w1_ref, a1_ref, c1_ref, s1_ref, s2_ref,
                    s1a, s2a):
    i = pl.program_id(0)
    sp = _spatial(nx_ref, q_ref)
    y1 = _dot(sp, w1_ref[...], ((1,), (1,)))
    h = jnp.maximum(a1_ref[...] * y1 + c1_ref[...], 0.0)

    @pl.when(i == 0)
    def _():
        s1a[...] = jnp.zeros_like(s1a)
        s2a[...] = jnp.zeros_like(s2a)

    s1a[...] += jnp.sum(h, axis=0, keepdims=True)
    s2a[...] += _dot(h, h, ((0,), (0,)))

    @pl.when(i == pl.num_programs(0) - 1)
    def _():
        s1_ref[...] = s1a[...]
        s2_ref[...] = s2a[...]


def _concat_scores(g_ref, q_ref, w1_ref, a1_ref, c1_ref, w2_ref,
                   a2_ref, c2_ref, ws_ref):
    g = g_ref[...]
    q = q_ref[...]
    nx = g[:, 64:67]
    rel = nx - q
    dist = jnp.sum(rel * rel, axis=1, keepdims=True)
    sp = jnp.concatenate([q, nx, rel, dist], axis=1)
    y1 = _dot(sp, w1_ref[...], ((1,), (1,)))
    h = jnp.maximum(a1_ref[...] * y1 + c1_ref[...], 0.0)
    y2 = _dot(h, w2_ref[...], ((1,), (1,)))
    eg = jnp.maximum(a2_ref[...] * y2 + c2_ref[...], 0.0)
    cat = jnp.concatenate([eg, g[:, 0:64]], axis=1)
    s = _dot(cat, ws_ref[...], ((1,), (1,)))
    return cat, s


def _smax_kernel(g_ref, q_ref, w1_ref, a1_ref, c1_ref, w2_ref,
                 a2_ref, c2_ref, ws_ref, sm_ref, ss_ref, macc, sacc):
    nt = pl.program_id(1)
    _, s = _concat_scores(g_ref, q_ref, w1_ref, a1_ref, c1_ref,
                          w2_ref, a2_ref, c2_ref, ws_ref)
    s3 = s.reshape(QT, K, D_AP)

    @pl.when(nt == 0)
    def _():
        macc[...] = jnp.full_like(macc, -1e30)
        sacc[...] = jnp.zeros_like(sacc)

    mold = macc[...]
    mnew = jnp.maximum(mold, jnp.max(s3, axis=0))
    e = jnp.exp(s3 - mnew[None, :, :])
    sacc[...] = sacc[...] * jnp.exp(mold - mnew) + jnp.sum(e, axis=0)
    macc[...] = mnew

    @pl.when(nt == pl.num_programs(1) - 1)
    def _():
        sm_ref[0] = macc[...]
        ss_ref[0] = sacc[...]


def _pool_kernel(g_ref, q_ref, w1_ref, a1_ref, c1_ref, w2_ref,
                 a2_ref, c2_ref, ws_ref, sm_ref, ss_ref,
                 feat_ref, f1_ref, f2_ref, f1a, f2a):
    b = pl.program_id(0)
    nt = pl.program_id(1)
    cat, s = _concat_scores(g_ref, q_ref, w1_ref, a1_ref, c1_ref,
                            w2_ref, a2_ref, c2_ref, ws_ref)
    s3 = s.reshape(QT, K, D_AP)
    c3 = cat.reshape(QT, K, D_AP)
    w = jnp.exp(s3 - sm_ref[0][None, :, :]) / ss_ref[0][None, :, :]
    feat = jnp.sum(c3 * w, axis=1)

    @pl.when((b == 0) & (nt == 0))
    def _():
        f1a[...] = jnp.zeros_like(f1a)
        f2a[...] = jnp.zeros_like(f2a)

    f1a[...] += jnp.sum(feat, axis=0, keepdims=True)
    f2a[...] += _dot(feat, feat, ((0,), (0,)))
    feat_ref[...] = feat

    @pl.when((b == B - 1) & (nt == pl.num_programs(1) - 1))
    def _():
        f1_ref[...] = f1a[...]
        f2_ref[...] = f2a[...]


def _head_kernel(feat_ref, x_ref, wm_ref, a3_ref, c3_ref, wsh_ref, as_ref,
                 cs_ref, out_ref):
    y = _dot(feat_ref[...], wm_ref[...], ((1,), (1,)))
    out = jnp.maximum(a3_ref[...] * y + c3_ref[...], 0.0)
    sc = as_ref[...] * _dot(x_ref[...], wsh_ref[...], ((1,), (1,))) + cs_ref[...]
    out_ref[...] = jnp.maximum(out + sc, 0.0)


def _full(shape):
    return pl.BlockSpec(shape, lambda *_: tuple(0 for _ in shape))


def kernel(xyz, features, W1, g1, b1, W2, g2, b2, Wscore, Wmlp, g3, b3,
           Wshort, gs, bs):
    f32 = jnp.float32
    xyz = xyz.astype(f32)
    feats2d = features.reshape(MROWS, D_IN)

    idx = _knn(xyz)
    tab = jnp.concatenate(
        [feats2d, xyz.reshape(MROWS, 3),
         jnp.zeros((MROWS, D_AP - D_IN - 3), f32)], axis=1)
    gat = _sc_gather(tab, idx.reshape(ROWS // 128, 128))
    nxg = gat[:, 64:80]

    qrep = jnp.broadcast_to(xyz[:, :, None, :], (B, N, K, 3)).reshape(ROWS, 3)

    fs1, fs2 = _moments(feats2d, 2048)
    a_s, c_s = _bn_fold(fs1, fs2, MROWS, Wshort, gs, bs)

    sp1, sp2 = pl.pallas_call(
        _sp_stats_kernel,
        grid=(ROWS // TG,),
        in_specs=[pl.BlockSpec((TG, 16), lambda i: (i, 0)),
                  pl.BlockSpec((TG, 3), lambda i: (i, 0))],
        out_specs=[_full((1, 10)), _full((10, 10))],
        out_shape=[jax.ShapeDtypeStruct((1, 10), f32),
                   jax.ShapeDtypeStruct((10, 10), f32)],
        scratch_shapes=[pltpu.VMEM((1, 10), f32), pltpu.VMEM((10, 10), f32)],
    )(nxg, qrep)
    a1, c1 = _bn_fold(sp1, sp2, ROWS, W1, g1, b1)

    h1, h2 = pl.pallas_call(
        _h_stats_kernel,
        grid=(ROWS // TG,),
        in_specs=[pl.BlockSpec((TG, 16), lambda i: (i, 0)),
                  pl.BlockSpec((TG, 3), lambda i: (i, 0)),
                  _full((D_MID, 10)), _full((1, D_MID)), _full((1, D_MID))],
        out_specs=[_full((1, D_MID)), _full((D_MID, D_MID))],
        out_shape=[jax.ShapeDtypeStruct((1, D_MID), f32),
                   jax.ShapeDtypeStruct((D_MID, D_MID), f32)],
        scratch_shapes=[pltpu.VMEM((1, D_MID), f32),
                        pltpu.VMEM((D_MID, D_MID), f32)],
    )(nxg, qrep, W1, a1, c1)
    a2, c2 = _bn_fold(h1, h2, ROWS, W2, g2, b2)

    gspecs = [pl.BlockSpec((TG, D_AP), lambda b_, t: (b_ * NTG + t, 0)),
              pl.BlockSpec((TG, 3), lambda b_, t: (b_ * NTG + t, 0)),
              _full((D_MID, 10)), _full((1, D_MID)), _full((1, D_MID)),
              _full((D_MID, D_MID)), _full((1, D_MID)), _full((1, D_MID)),
              _full((D_AP, D_AP))]
    sm, ss = pl.pallas_call(
        _smax_kernel,
        grid=(B, NTG),
        in_specs=gspecs,
        out_specs=[pl.BlockSpec((1, K, D_AP), lambda b_, t: (b_, 0, 0))] * 2,
        out_shape=[jax.ShapeDtypeStruct((B, K, D_AP), f32)] * 2,
        scratch_shapes=[pltpu.VMEM((K, D_AP), f32),
                        pltpu.VMEM((K, D_AP), f32)],
    )(gat, qrep, W1, a1, c1, W2, a2, c2, Wscore)

    feat, f1, f2 = pl.pallas_call(
        _pool_kernel,
        grid=(B, NTG),
        in_specs=gspecs + [
            pl.BlockSpec((1, K, D_AP), lambda b_, t: (b_, 0, 0)),
            pl.BlockSpec((1, K, D_AP), lambda b_, t: (b_, 0, 0))],
        out_specs=[pl.BlockSpec((QT, D_AP), lambda b_, t: (b_ * NTG + t, 0)),
                   _full((1, D_AP)), _full((D_AP, D_AP))],
        out_shape=[jax.ShapeDtypeStruct((MROWS, D_AP), f32),
                   jax.ShapeDtypeStruct((1, D_AP), f32),
                   jax.ShapeDtypeStruct((D_AP, D_AP), f32)],
        scratch_shapes=[pltpu.VMEM((1, D_AP), f32),
                        pltpu.VMEM((D_AP, D_AP), f32)],
    )(gat, qrep, W1, a1, c1, W2, a2, c2, Wscore, sm, ss)
    a3, c3 = _bn_fold(f1, f2, MROWS, Wmlp, g3, b3)

    out = pl.pallas_call(
        _head_kernel,
        grid=(MROWS // 2048,),
        in_specs=[pl.BlockSpec((2048, D_AP), lambda i: (i, 0)),
                  pl.BlockSpec((2048, D_IN), lambda i: (i, 0)),
                  _full((D_OUT, D_AP)), _full((1, D_OUT)), _full((1, D_OUT)),
                  _full((D_OUT, D_IN)), _full((1, D_OUT)), _full((1, D_OUT))],
        out_specs=pl.BlockSpec((2048, D_OUT), lambda i: (i, 0)),
        out_shape=jax.ShapeDtypeStruct((MROWS, D_OUT), f32),
    )(feat, feats2d, Wmlp, a3, c3, Wshort, a_s, c_s)
    return out.reshape(B, N, D_OUT)


# speedup vs baseline: 9.9032x; 1.8611x over previous
import functools

import jax
import jax.numpy as jnp
from jax import lax
from jax.experimental import pallas as pl
from jax.experimental.pallas import tpu as pltpu
from jax.experimental.pallas import tpu_sc as plsc

B, N, K = 4, 4096, 16
D_IN, D_OUT = 64, 128
D_MID = D_OUT // 2
D_AP = D_MID + D_IN
EPS = 1e-5

TQ = 256
NTQ = N // TQ
TG = 2048
QT = TG // K
NTG = N // QT
ROWS = B * N * K
MROWS = B * N

_HIGH = lax.Precision.HIGHEST


def _dot(a, b, dims):
    return lax.dot_general(a, b, (dims, ((), ())), precision=_HIGH,
                           preferred_element_type=jnp.float32)


def _dotf(a, b, dims):
    return lax.dot_general(a, b, (dims, ((), ())),
                           preferred_element_type=jnp.float32)


def _knn_kernel(q_ref, a_ref, sqq_ref, sqa_ref, idx_ref):
    b = pl.program_id(0)
    cross = lax.dot_general(q_ref[0, 0], a_ref[0, 0], (((1,), (1,)), ((), ())),
                            preferred_element_type=jnp.float32)
    d = sqq_ref[0, 0] + sqa_ref[0] - 2.0 * cross
    col = lax.broadcasted_iota(jnp.int32, (TQ, N), 1)
    base = (b * N).astype(jnp.int32)
    picks = []
    for _ in range(K):
        m = jnp.min(d, axis=1, keepdims=True)
        am = jnp.min(jnp.where(d == m, col, N), axis=1, keepdims=True)
        picks.append(am + base)
        d = jnp.where(col == am, jnp.float32(jnp.inf), d)
    idx_ref[0, 0] = jnp.concatenate(picks, axis=1)


def _knn(xyz):
    sq = jnp.sum(xyz * xyz, -1)
    out = pl.pallas_call(
        _knn_kernel,
        grid=(B, NTQ),
        in_specs=[pl.BlockSpec((1, 1, TQ, 3), lambda b, t: (b, t, 0, 0)),
                  pl.BlockSpec((1, 1, N, 3), lambda b, t: (b, 0, 0, 0)),
                  pl.BlockSpec((1, 1, TQ, 1), lambda b, t: (b, t, 0, 0)),
                  pl.BlockSpec((1, 1, N), lambda b, t: (b, 0, 0))],
        out_specs=pl.BlockSpec((1, 1, TQ, K), lambda b, t: (b, t, 0, 0)),
        out_shape=jax.ShapeDtypeStruct((B, NTQ, TQ, K), jnp.int32),
    )(xyz.reshape(B, NTQ, TQ, 3), xyz.reshape(B, 1, N, 3),
      sq.reshape(B, NTQ, TQ, 1), sq.reshape(B, 1, N))
    return out.reshape(ROWS)


_CH = 128
_RPW = ROWS // 32
_NCH = _RPW // _CH


def _sc_gather(tab, idx2d):
    info = plsc.get_sparse_core_info()
    nc = info.num_cores
    mesh = plsc.VectorSubcoreMesh(core_axis_name="c", subcore_axis_name="s")

    @functools.partial(
        pl.kernel,
        mesh=mesh,
        out_type=jax.ShapeDtypeStruct((ROWS, D_AP), jnp.float32),
        scratch_types=[
            pltpu.VMEM((_NCH, _CH), jnp.int32),
            pltpu.VMEM((2, _CH, D_AP), jnp.float32),
            pltpu.SemaphoreType.DMA,
            pltpu.SemaphoreType.DMA,
        ],
    )
    def gather_k(tab_hbm, idx_hbm, out_f, idx_v, fv, s1, s2):
        wid = lax.axis_index("s") * nc + lax.axis_index("c")
        pltpu.sync_copy(idx_hbm.at[pl.ds(wid * _NCH, _NCH)], idx_v)
        base = wid * _RPW
        pltpu.async_copy(tab_hbm.at[idx_v.at[0]], fv.at[0], s1)

        @pl.loop(0, _NCH, step=2)
        def _chunk(c):
            pltpu.async_copy(tab_hbm.at[idx_v.at[c + 1]], fv.at[1], s2)
            pltpu.make_async_copy(tab_hbm.at[pl.ds(0, _CH)], fv.at[0],
                                  s1).wait()
            pltpu.sync_copy(fv.at[0], out_f.at[pl.ds(base + c * _CH, _CH)])

            @pl.when(c + 2 < _NCH)
            def _():
                pltpu.async_copy(tab_hbm.at[idx_v.at[c + 2]], fv.at[0], s1)

            pltpu.make_async_copy(tab_hbm.at[pl.ds(0, _CH)], fv.at[1],
                                  s2).wait()
            pltpu.sync_copy(fv.at[1],
                            out_f.at[pl.ds(base + (c + 1) * _CH, _CH)])

    return gather_k(tab, idx2d)


def _stats_kernel(x_ref, s1_ref, s2_ref, s1a, s2a):
    i = pl.program_id(0)
    x = x_ref[...]

    @pl.when(i == 0)
    def _():
        s1a[...] = jnp.zeros_like(s1a)
        s2a[...] = jnp.zeros_like(s2a)

    s1a[...] += jnp.sum(x, axis=0, keepdims=True)
    s2a[...] += _dot(x, x, ((0,), (0,)))

    @pl.when(i == pl.num_programs(0) - 1)
    def _():
        s1_ref[...] = s1a[...]
        s2_ref[...] = s2a[...]


def _moments(x2d, tile):
    rows, d = x2d.shape
    grid = rows // tile
    s1, s2 = pl.pallas_call(
        _stats_kernel,
        grid=(grid,),
        in_specs=[pl.BlockSpec((tile, d), lambda i: (i, 0))],
        out_specs=[pl.BlockSpec((1, d), lambda i: (0, 0)),
                   pl.BlockSpec((d, d), lambda i: (0, 0))],
        out_shape=[jax.ShapeDtypeStruct((1, d), jnp.float32),
                   jax.ShapeDtypeStruct((d, d), jnp.float32)],
        scratch_shapes=[pltpu.VMEM((1, d), jnp.float32),
                        pltpu.VMEM((d, d), jnp.float32)],
    )(x2d)
    return s1, s2


def _bn_fold(s1, s2, m_count, w, g, bias):
    mean_x = s1[0] / m_count
    mean_y = w @ mean_x
    ey2 = jnp.einsum("ij,jk,ik->i", w, s2 / m_count, w)
    var_y = ey2 - mean_y * mean_y
    a = g / jnp.sqrt(var_y + EPS)
    c = bias - a * mean_y
    return a.reshape(1, -1), c.reshape(1, -1)


def _spatial(nx_ref, q_ref):
    nx = nx_ref[...][:, 0:3]
    q = q_ref[...]
    rel = nx - q
    dist = jnp.sum(rel * rel, axis=1, keepdims=True)
    return jnp.concatenate([q, nx, rel, dist], axis=1)


def _sp_stats_kernel(nx_ref, q_ref, s1_ref, s2_ref, s1a, s2a):
    i = pl.program_id(0)
    sp = _spatial(nx_ref, q_ref)

    @pl.when(i == 0)
    def _():
        s1a[...] = jnp.zeros_like(s1a)
        s2a[...] = jnp.zeros_like(s2a)

    s1a[...] += jnp.sum(sp, axis=0, keepdims=True)
    s2a[...] += _dot(sp, sp, ((0,), (0,)))

    @pl.when(i == pl.num_programs(0) - 1)
    def _():
        s1_ref[...] = s1a[...]
        s2_ref[...] = s2a[...]


def _h_stats_kernel(nx_ref, q_ref, w1_ref, a1_ref, c1_ref, s1_ref, s2_ref,
                    s1a, s2a):
    i = pl.program_id(0)
    sp = _spatial(nx_ref, q_ref)
    y1 = _dotf(sp, w1_ref[...], ((1,), (1,)))
    h = jnp.maximum(a1_ref[...] * y1 + c1_ref[...], 0.0)

    @pl.when(i == 0)
    def _():
        s1a[...] = jnp.zeros_like(s1a)
        s2a[...] = jnp.zeros_like(s2a)

    s1a[...] += jnp.sum(h, axis=0, keepdims=True)
    s2a[...] += _dot(h, h, ((0,), (0,)))

    @pl.when(i == pl.num_programs(0) - 1)
    def _():
        s1_ref[...] = s1a[...]
        s2_ref[...] = s2a[...]


def _concat_scores(g_ref, q_ref, w1_ref, a1_ref, c1_ref, w2_ref,
                   a2_ref, c2_ref, ws_ref):
    g = g_ref[...]
    q = q_ref[...]
    nx = g[:, 64:67]
    rel = nx - q
    dist = jnp.sum(rel * rel, axis=1, keepdims=True)
    sp = jnp.concatenate([q, nx, rel, dist], axis=1)
    y1 = _dotf(sp, w1_ref[...], ((1,), (1,)))
    h = jnp.maximum(a1_ref[...] * y1 + c1_ref[...], 0.0)
    y2 = _dotf(h, w2_ref[...], ((1,), (1,)))
    eg = jnp.maximum(a2_ref[...] * y2 + c2_ref[...], 0.0)
    cat = jnp.concatenate([eg, g[:, 0:64]], axis=1)
    s = _dotf(cat, ws_ref[...], ((1,), (1,)))
    return cat, s


def _smax_kernel(g_ref, q_ref, w1_ref, a1_ref, c1_ref, w2_ref,
                 a2_ref, c2_ref, ws_ref, sm_ref, ss_ref, macc, sacc):
    nt = pl.program_id(1)
    _, s = _concat_scores(g_ref, q_ref, w1_ref, a1_ref, c1_ref,
                          w2_ref, a2_ref, c2_ref, ws_ref)
    s3 = s.reshape(QT, K, D_AP)

    @pl.when(nt == 0)
    def _():
        macc[...] = jnp.full_like(macc, -1e30)
        sacc[...] = jnp.zeros_like(sacc)

    mold = macc[...]
    mnew = jnp.maximum(mold, jnp.max(s3, axis=0))
    e = jnp.exp(s3 - mnew[None, :, :])
    sacc[...] = sacc[...] * jnp.exp(mold - mnew) + jnp.sum(e, axis=0)
    macc[...] = mnew

    @pl.when(nt == pl.num_programs(1) - 1)
    def _():
        sm_ref[0] = macc[...]
        ss_ref[0] = sacc[...]


def _pool_kernel(g_ref, q_ref, w1_ref, a1_ref, c1_ref, w2_ref,
                 a2_ref, c2_ref, ws_ref, sm_ref, ss_ref,
                 feat_ref, f1_ref, f2_ref, f1a, f2a):
    b = pl.program_id(0)
    nt = pl.program_id(1)
    cat, s = _concat_scores(g_ref, q_ref, w1_ref, a1_ref, c1_ref,
                            w2_ref, a2_ref, c2_ref, ws_ref)
    s3 = s.reshape(QT, K, D_AP)
    c3 = cat.reshape(QT, K, D_AP)
    w = jnp.exp(s3 - sm_ref[0][None, :, :]) / ss_ref[0][None, :, :]
    feat = jnp.sum(c3 * w, axis=1)

    @pl.when((b == 0) & (nt == 0))
    def _():
        f1a[...] = jnp.zeros_like(f1a)
        f2a[...] = jnp.zeros_like(f2a)

    f1a[...] += jnp.sum(feat, axis=0, keepdims=True)
    f2a[...] += _dot(feat, feat, ((0,), (0,)))
    feat_ref[...] = feat

    @pl.when((b == B - 1) & (nt == pl.num_programs(1) - 1))
    def _():
        f1_ref[...] = f1a[...]
        f2_ref[...] = f2a[...]


def _head_kernel(feat_ref, x_ref, wm_ref, a3_ref, c3_ref, wsh_ref, as_ref,
                 cs_ref, out_ref):
    y = _dotf(feat_ref[...], wm_ref[...], ((1,), (1,)))
    out = jnp.maximum(a3_ref[...] * y + c3_ref[...], 0.0)
    sc = as_ref[...] * _dotf(x_ref[...], wsh_ref[...], ((1,), (1,))) + cs_ref[...]
    out_ref[...] = jnp.maximum(out + sc, 0.0)


def _full(shape):
    return pl.BlockSpec(shape, lambda *_: tuple(0 for _ in shape))


def kernel(xyz, features, W1, g1, b1, W2, g2, b2, Wscore, Wmlp, g3, b3,
           Wshort, gs, bs):
    f32 = jnp.float32
    xyz = xyz.astype(f32)
    feats2d = features.reshape(MROWS, D_IN)

    idx = _knn(xyz)
    tab = jnp.concatenate(
        [feats2d, xyz.reshape(MROWS, 3),
         jnp.zeros((MROWS, D_AP - D_IN - 3), f32)], axis=1)
    gat = _sc_gather(tab, idx.reshape(ROWS // 128, 128))
    nxg = gat[:, 64:80]

    qrep = jnp.broadcast_to(xyz[:, :, None, :], (B, N, K, 3)).reshape(ROWS, 3)

    fs1, fs2 = _moments(feats2d, 2048)
    a_s, c_s = _bn_fold(fs1, fs2, MROWS, Wshort, gs, bs)

    sp1, sp2 = pl.pallas_call(
        _sp_stats_kernel,
        grid=(ROWS // TG,),
        in_specs=[pl.BlockSpec((TG, 16), lambda i: (i, 0)),
                  pl.BlockSpec((TG, 3), lambda i: (i, 0))],
        out_specs=[_full((1, 10)), _full((10, 10))],
        out_shape=[jax.ShapeDtypeStruct((1, 10), f32),
                   jax.ShapeDtypeStruct((10, 10), f32)],
        scratch_shapes=[pltpu.VMEM((1, 10), f32), pltpu.VMEM((10, 10), f32)],
    )(nxg, qrep)
    a1, c1 = _bn_fold(sp1, sp2, ROWS, W1, g1, b1)

    h1, h2 = pl.pallas_call(
        _h_stats_kernel,
        grid=(ROWS // TG,),
        in_specs=[pl.BlockSpec((TG, 16), lambda i: (i, 0)),
                  pl.BlockSpec((TG, 3), lambda i: (i, 0)),
                  _full((D_MID, 10)), _full((1, D_MID)), _full((1, D_MID))],
        out_specs=[_full((1, D_MID)), _full((D_MID, D_MID))],
        out_shape=[jax.ShapeDtypeStruct((1, D_MID), f32),
                   jax.ShapeDtypeStruct((D_MID, D_MID), f32)],
        scratch_shapes=[pltpu.VMEM((1, D_MID), f32),
                        pltpu.VMEM((D_MID, D_MID), f32)],
    )(nxg, qrep, W1, a1, c1)
    a2, c2 = _bn_fold(h1, h2, ROWS, W2, g2, b2)

    gspecs = [pl.BlockSpec((TG, D_AP), lambda b_, t: (b_ * NTG + t, 0)),
              pl.BlockSpec((TG, 3), lambda b_, t: (b_ * NTG + t, 0)),
              _full((D_MID, 10)), _full((1, D_MID)), _full((1, D_MID)),
              _full((D_MID, D_MID)), _full((1, D_MID)), _full((1, D_MID)),
              _full((D_AP, D_AP))]
    sm, ss = pl.pallas_call(
        _smax_kernel,
        grid=(B, NTG),
        in_specs=gspecs,
        out_specs=[pl.BlockSpec((1, K, D_AP), lambda b_, t: (b_, 0, 0))] * 2,
        out_shape=[jax.ShapeDtypeStruct((B, K, D_AP), f32)] * 2,
        scratch_shapes=[pltpu.VMEM((K, D_AP), f32),
                        pltpu.VMEM((K, D_AP), f32)],
    )(gat, qrep, W1, a1, c1, W2, a2, c2, Wscore)

    feat, f1, f2 = pl.pallas_call(
        _pool_kernel,
        grid=(B, NTG),
        in_specs=gspecs + [
            pl.BlockSpec((1, K, D_AP), lambda b_, t: (b_, 0, 0)),
            pl.BlockSpec((1, K, D_AP), lambda b_, t: (b_, 0, 0))],
        out_specs=[pl.BlockSpec((QT, D_AP), lambda b_, t: (b_ * NTG + t, 0)),
                   _full((1, D_AP)), _full((D_AP, D_AP))],
        out_shape=[jax.ShapeDtypeStruct((MROWS, D_AP), f32),
                   jax.ShapeDtypeStruct((1, D_AP), f32),
                   jax.ShapeDtypeStruct((D_AP, D_AP), f32)],
        scratch_shapes=[pltpu.VMEM((1, D_AP), f32),
                        pltpu.VMEM((D_AP, D_AP), f32)],
    )(gat, qrep, W1, a1, c1, W2, a2, c2, Wscore, sm, ss)
    a3, c3 = _bn_fold(f1, f2, MROWS, Wmlp, g3, b3)

    out = pl.pallas_call(
        _head_kernel,
        grid=(MROWS // 2048,),
        in_specs=[pl.BlockSpec((2048, D_AP), lambda i: (i, 0)),
                  pl.BlockSpec((2048, D_IN), lambda i: (i, 0)),
                  _full((D_OUT, D_AP)), _full((1, D_OUT)), _full((1, D_OUT)),
                  _full((D_OUT, D_IN)), _full((1, D_OUT)), _full((1, D_OUT))],
        out_specs=pl.BlockSpec((2048, D_OUT), lambda i: (i, 0)),
        out_shape=jax.ShapeDtypeStruct((MROWS, D_OUT), f32),
    )(feat, feats2d, Wmlp, a3, c3, Wshort, a_s, c_s)
    return out.reshape(B, N, D_OUT)
